# Initial kernel scaffold; baseline (speedup 1.0000x reference)
import jax
import jax.numpy as jnp
from jax.experimental import pallas as pl


def kernel(x, edge_index, batch, W0, b0, W1, b1, Wn, bn, We, be):
    raise NotImplementedError("write your pallas kernel here")


# speedup vs baseline: 10.1839x; 10.1839x over previous
import jax
import jax.numpy as jnp
from jax import lax
from jax.experimental import pallas as pl
from jax.experimental.pallas import tpu as pltpu
from jax.experimental.pallas import tpu_sc as plsc

_N = 10000
_E = 160000
_D = 256
_H = 256
_G = 128
_NP = 10240
_EP = 163840
_CHN = 128
_ROWS = _EP // _CHN
_TILES = 16
_HALF = _H // 2
_RPT = _NP // _TILES
_BR = 1024

_MESH = plsc.VectorSubcoreMesh(core_axis_name="c", subcore_axis_name="s")


def _deg_body(d2d, zeros_c, ones_c, deg2, didx, buf, deg_sp):
    cid = lax.axis_index("c")
    sid = lax.axis_index("s")
    nrow_t = _ROWS // 2 // _TILES
    pltpu.sync_copy(d2d.at[pl.ds(cid * (_ROWS // 2) + sid * nrow_t, nrow_t), :],
                    didx)
    pltpu.sync_copy(zeros_c, buf)

    def zinit(i, _):
        pltpu.sync_copy(buf, deg_sp.at[pl.ds(sid * _RPT + i * _CHN, _CHN), :])
        return _
    lax.fori_loop(0, _RPT // _CHN, zinit, None)
    pltpu.sync_copy(ones_c, buf)
    plsc.subcore_barrier()

    def add1(c, _):
        pltpu.sync_copy(buf, deg_sp.at[didx.at[c]], add=True)
        return _
    lax.fori_loop(0, nrow_t, add1, None)
    plsc.subcore_barrier()

    def rd(i, _):
        r = sid * _RPT + i * _CHN
        pltpu.sync_copy(deg_sp.at[pl.ds(r, _CHN), :], buf)
        pltpu.sync_copy(buf, deg2.at[cid].at[pl.ds(r, _CHN), :])
        return _
    lax.fori_loop(0, _RPT // _CHN, rd, None)


_deg_call = pl.kernel(
    _deg_body,
    out_type=jax.ShapeDtypeStruct((2, _NP, _HALF), jnp.float32),
    mesh=_MESH,
    scratch_types=[
        pltpu.VMEM((_ROWS // 2 // _TILES, _CHN), jnp.int32),
        pltpu.VMEM((_CHN, _HALF), jnp.float32),
        pltpu.VMEM_SHARED((_NP, _HALF), jnp.float32),
    ],
)


def _mp_body(s2d, d2d, y3, z3, sidx, didx, rows, z_sp):
    cid = lax.axis_index("c")
    sid = lax.axis_index("s")
    nrow_t = _ROWS // _TILES
    pltpu.sync_copy(s2d.at[pl.ds(sid * nrow_t, nrow_t), :], sidx)
    pltpu.sync_copy(d2d.at[pl.ds(sid * nrow_t, nrow_t), :], didx)

    def zinit(i, _):
        r = sid * _RPT + i * _CHN
        pltpu.sync_copy(y3.at[cid].at[pl.ds(r, _CHN), :], rows)
        pltpu.sync_copy(rows, z_sp.at[pl.ds(r, _CHN), :])
        return _
    lax.fori_loop(0, _RPT // _CHN, zinit, None)
    plsc.subcore_barrier()

    def body(c, _):
        pltpu.sync_copy(y3.at[cid].at[sidx.at[c]], rows)
        pltpu.sync_copy(rows, z_sp.at[didx.at[c]], add=True)
        return _
    lax.fori_loop(0, nrow_t, body, None)
    plsc.subcore_barrier()

    def rd(i, _):
        r = sid * _RPT + i * _CHN
        pltpu.sync_copy(z_sp.at[pl.ds(r, _CHN), :], rows)
        pltpu.sync_copy(rows, z3.at[cid].at[pl.ds(r, _CHN), :])
        return _
    lax.fori_loop(0, _RPT // _CHN, rd, None)


_mp_call = pl.kernel(
    _mp_body,
    out_type=jax.ShapeDtypeStruct((2, _NP, _HALF), jnp.float32),
    mesh=_MESH,
    scratch_types=[
        pltpu.VMEM((_ROWS // _TILES, _CHN), jnp.int32),
        pltpu.VMEM((_ROWS // _TILES, _CHN), jnp.int32),
        pltpu.VMEM((_CHN, _HALF), jnp.float32),
        pltpu.VMEM_SHARED((_NP, _HALF), jnp.float32),
    ],
)


def _eg_body(s2d, d2d, t, u, ex, sidx, didx, tv, uv, stage):
    cid = lax.axis_index("c")
    sid = lax.axis_index("s")
    wid = sid * 2 + cid
    nrow_w = _ROWS // 32
    epw = nrow_w * _CHN
    pltpu.sync_copy(s2d.at[pl.ds(wid * nrow_w, nrow_w), :], sidx)
    pltpu.sync_copy(d2d.at[pl.ds(wid * nrow_w, nrow_w), :], didx)
    pltpu.sync_copy(t, tv)
    pltpu.sync_copy(u, uv)
    lane = lax.iota(jnp.int32, 16)

    def chunk(c, _):
        for k in range(_CHN // 16):
            s = sidx[c, pl.ds(k * 16, 16)] * 2
            d = didx[c, pl.ds(k * 16, 16)] * 2
            t0 = plsc.load_gather(tv, [s])
            t1 = plsc.load_gather(tv, [s + 1])
            u0 = plsc.load_gather(uv, [d])
            u1 = plsc.load_gather(uv, [d + 1])
            le = (c * _CHN + k * 16 + lane) * 2
            plsc.store_scatter(stage, [le], t0 + u0)
            plsc.store_scatter(stage, [le + 1], t1 + u1)
        return _
    lax.fori_loop(0, nrow_w, chunk, None)
    pltpu.sync_copy(stage, ex.at[pl.ds(wid * epw * 2, epw * 2)])


_eg_call = pl.kernel(
    _eg_body,
    out_type=jax.ShapeDtypeStruct((_EP * 2,), jnp.float32),
    mesh=_MESH,
    scratch_types=[
        pltpu.VMEM((_ROWS // 32, _CHN), jnp.int32),
        pltpu.VMEM((_ROWS // 32, _CHN), jnp.int32),
        pltpu.VMEM((_NP * 2,), jnp.float32),
        pltpu.VMEM((_NP * 2,), jnp.float32),
        pltpu.VMEM((_EP // 32 * 2,), jnp.float32),
    ],
    compiler_params=pltpu.CompilerParams(needs_layout_passes=False),
)


def _dis_of(deg_ref):
    return lax.rsqrt(deg_ref[0, :, 0:1] + deg_ref[1, :, 0:1] + 1.0)


def _s1_body(x_ref, w_ref, deg_ref, y_ref):
    dis = _dis_of(deg_ref)
    xw = jnp.dot(x_ref[...], w_ref[...], preferred_element_type=jnp.float32)
    y = xw * dis
    y_ref[0] = y[:, :_HALF]
    y_ref[1] = y[:, _HALF:]


def _s3_body(z_ref, deg_ref, b0_ref, w_ref, y_ref):
    dis = _dis_of(deg_ref)
    z = jnp.concatenate([z_ref[0], z_ref[1]], axis=1)
    h = jnp.maximum(z * dis + b0_ref[...], 0.0)
    y = jnp.dot(h, w_ref[...], preferred_element_type=jnp.float32) * dis
    y_ref[0] = y[:, :_HALF]
    y_ref[1] = y[:, _HALF:]


def _s5_body(z_ref, deg_ref, b1_ref, wet_ref, web_ref,
             be_ref, wn_ref, bn_ref, batch_ref,
             t_ref, u_ref, nx_ref, sums_scr, cnt_scr):
    i = pl.program_id(0)
    dis = _dis_of(deg_ref)
    emb = jnp.concatenate([z_ref[0], z_ref[1]], axis=1) * dis + b1_ref[...]
    t_ref[...] = (jnp.dot(emb, wet_ref[...], preferred_element_type=jnp.float32)
                  + be_ref[...])
    u_ref[...] = jnp.dot(emb, web_ref[...], preferred_element_type=jnp.float32)
    onehot = (batch_ref[...] ==
              lax.broadcasted_iota(jnp.int32, (1, _G), 1)).astype(jnp.float32)
    sums_part = lax.dot_general(onehot, emb, (((0,), (0,)), ((), ())),
                                preferred_element_type=jnp.float32)
    ones = jnp.ones((_BR, 8), jnp.float32)
    cnt_part = lax.dot_general(onehot, ones, (((0,), (0,)), ((), ())),
                               preferred_element_type=jnp.float32)

    @pl.when(i == 0)
    def _init():
        sums_scr[...] = sums_part
        cnt_scr[...] = cnt_part

    @pl.when(i > 0)
    def _acc():
        sums_scr[...] += sums_part
        cnt_scr[...] += cnt_part

    @pl.when(i == pl.num_programs(0) - 1)
    def _fin():
        pooled = sums_scr[...] / jnp.maximum(cnt_scr[...][:, 0:1], 1.0)
        nx_ref[...] = jnp.dot(pooled, wn_ref[...],
                              preferred_element_type=jnp.float32) + bn_ref[...]


def _row_spec(w):
    return pl.BlockSpec((_BR, w), lambda r: (r, 0))


def _half_spec():
    return pl.BlockSpec((2, _BR, _HALF), lambda r: (0, r, 0))


def _deg_spec():
    return pl.BlockSpec((2, _BR, _HALF), lambda r: (0, r, 0))


def _full_spec(shape):
    return pl.BlockSpec(shape, lambda r: tuple(0 for _ in shape))


def _s1_call(x, W0, deg2):
    return pl.pallas_call(
        _s1_body,
        grid=(_NP // _BR,),
        in_specs=[_row_spec(_D), _full_spec((_D, _H)), _deg_spec()],
        out_specs=_half_spec(),
        out_shape=jax.ShapeDtypeStruct((2, _NP, _HALF), jnp.float32),
    )(x, W0, deg2)


def _s3_call(z3, deg2, b0, W1):
    return pl.pallas_call(
        _s3_body,
        grid=(_NP // _BR,),
        in_specs=[_half_spec(), _deg_spec(),
                  _full_spec((1, _H)), _full_spec((_H, _H))],
        out_specs=_half_spec(),
        out_shape=jax.ShapeDtypeStruct((2, _NP, _HALF), jnp.float32),
    )(z3, deg2, b0, W1)


def _s5_call(z3, deg2, b1, Wet, Web, be, Wn, bn, batch_col):
    return pl.pallas_call(
        _s5_body,
        grid=(_NP // _BR,),
        in_specs=[_half_spec(), _deg_spec(),
                  _full_spec((1, _H)), _full_spec((_H, 2)), _full_spec((_H, 2)),
                  _full_spec((1, 2)), _full_spec((_H, 2)), _full_spec((1, 2)),
                  _row_spec(1)],
        out_specs=(_row_spec(2), _row_spec(2), _full_spec((_G, 2))),
        out_shape=(jax.ShapeDtypeStruct((_NP, 2), jnp.float32),
                   jax.ShapeDtypeStruct((_NP, 2), jnp.float32),
                   jax.ShapeDtypeStruct((_G, 2), jnp.float32)),
        scratch_shapes=[pltpu.VMEM((_G, _H), jnp.float32),
                        pltpu.VMEM((_G, 8), jnp.float32)],
    )(z3, deg2, b1, Wet, Web, be, Wn, bn, batch_col)


def kernel(x, edge_index, batch, W0, b0, W1, b1, Wn, bn, We, be):
    npad = _NP - _N
    epad = _EP - _E
    pad_ids = _N + (jnp.arange(epad, dtype=jnp.int32) % npad)
    s2d = jnp.concatenate([edge_index[0], pad_ids]).reshape(_ROWS, _CHN)
    d2d = jnp.concatenate([edge_index[1], pad_ids]).reshape(_ROWS, _CHN)
    x_p = jnp.pad(x, ((0, npad), (0, 0)))
    batch_p = jnp.pad(batch, (0, npad), constant_values=_G).reshape(-1, 1)
    zeros_c = jnp.zeros((_CHN, _HALF), jnp.float32)
    ones_c = jnp.ones((_CHN, _HALF), jnp.float32)

    deg2 = _deg_call(d2d, zeros_c, ones_c)
    y0 = _s1_call(x_p, W0, deg2)
    z0 = _mp_call(s2d, d2d, y0)
    y1 = _s3_call(z0, deg2, b0.reshape(1, -1), W1)
    z1 = _mp_call(s2d, d2d, y1)
    t, u, node_x = _s5_call(z1, deg2, b1.reshape(1, -1),
                            We[:_H], We[_H:], be.reshape(1, -1), Wn,
                            bn.reshape(1, -1), batch_p)
    edge_x = _eg_call(s2d, d2d, t.reshape(-1), u.reshape(-1))
    return node_x, edge_x.reshape(_EP, 2)[:_E]
